# Initial kernel scaffold; baseline (speedup 1.0000x reference)
import jax
import jax.numpy as jnp
from jax.experimental import pallas as pl


def kernel(in_feat, edge_index, W1, attn_l1, attn_r1, b1, W2, attn_l2, attn_r2, b2):
    raise NotImplementedError("write your pallas kernel here")


# speedup vs baseline: 15.7238x; 15.7238x over previous
import functools

import jax
import jax.numpy as jnp
import numpy as np
from jax import lax
from jax.experimental import pallas as pl
from jax.experimental.pallas import tpu as pltpu
from jax.experimental.pallas import tpu_sc as plsc

NC = 2
NS = 16
B = 128
NH = 8
FH = 64
D = NH * FH


def _mm_body(x_ref, w_ref, al_ref, ar_ref, f0, f1, f2, f3, alo, bro):
    feat = jnp.dot(x_ref[...], w_ref[...], preferred_element_type=jnp.float32)
    f0[...] = feat[:, 0:128]
    f1[...] = feat[:, 128:256]
    f2[...] = feat[:, 256:384]
    f3[...] = feat[:, 384:512]
    alo[...] = jnp.dot(feat, al_ref[...], preferred_element_type=jnp.float32)
    bro[...] = jnp.dot(feat, ar_ref[...], preferred_element_type=jnp.float32)


def _mm_call(x, Wm, Aexp_l, Aexp_r):
    n, k = x.shape
    bm = 1000
    f32 = jnp.float32
    return pl.pallas_call(
        _mm_body,
        grid=(n // bm,),
        in_specs=[
            pl.BlockSpec((bm, k), lambda i: (i, 0)),
            pl.BlockSpec((k, D), lambda i: (0, 0)),
            pl.BlockSpec((D, 128), lambda i: (0, 0)),
            pl.BlockSpec((D, 128), lambda i: (0, 0)),
        ],
        out_specs=[pl.BlockSpec((bm, 128), lambda i: (i, 0))] * 6,
        out_shape=[jax.ShapeDtypeStruct((n, 128), f32)] * 6,
    )(x, Wm, Aexp_l, Aexp_r)


def _partition(N, E):
    nb = E // (NC * NS * B)
    cov = NC * NS * B * nb
    rem_b = (E - cov) // B
    assert cov + rem_b * B == E
    rpt = (N // NS) // 8 * 8
    rlast = N - rpt * (NS - 1)
    return nb, cov, rem_b, rpt, rlast


def _striped(copy_fn, sid, rpt, rlast):
    @pl.when(sid < NS - 1)
    def _():
        copy_fn(sid * rpt, rpt)

    @pl.when(sid == NS - 1)
    def _():
        copy_fn((NS - 1) * rpt, rlast)


def _edge_loop(wid, nb, cov, rem_b, do_batch):
    base_w = wid * (nb * B)

    def bloop(j, carry):
        do_batch(base_w + j * B)
        return carry

    lax.fori_loop(0, nb, bloop, 0)
    if rem_b:
        @pl.when(wid < rem_b)
        def _():
            do_batch(cov + wid * B)


def _make_edge_kernel(N, E):
    nb, cov, rem_b, _, _ = _partition(N, E)
    f32 = jnp.float32
    mesh = plsc.VectorSubcoreMesh(core_axis_name="c", subcore_axis_name="s")

    @functools.partial(
        pl.kernel,
        out_type=jax.ShapeDtypeStruct((E, 128), f32),
        mesh=mesh,
        scratch_types=[
            pltpu.VMEM((B,), jnp.int32),
            pltpu.VMEM((B,), jnp.int32),
            pltpu.VMEM((B, 128), f32),
            pltpu.VMEM((B, 128), f32),
            pltpu.SemaphoreType.DMA,
            pltpu.SemaphoreType.DMA,
        ],
    )
    def ek(al_h, br_h, src_h, dst_h, w_h, isb, idb, ga, gb, sem1, sem2):
        cid = lax.axis_index("c")
        sid = lax.axis_index("s")
        wid = sid * NC + cid

        def do_batch(base):
            pltpu.sync_copy(src_h.at[pl.ds(base, B)], isb)
            pltpu.sync_copy(dst_h.at[pl.ds(base, B)], idb)
            cp1 = pltpu.async_copy(al_h.at[isb], ga, sem1)
            cp2 = pltpu.async_copy(br_h.at[idb], gb, sem2)
            cp1.wait()
            cp2.wait()

            def body(kk, carry):
                for jj in range(8):
                    sl = pl.ds(jj * 16, 16)
                    x = ga[kk, sl] + gb[kk, sl]
                    ga[kk, sl] = jnp.exp(jnp.maximum(x, 0.2 * x))
                return carry

            lax.fori_loop(0, B, body, 0)
            pltpu.sync_copy(ga, w_h.at[pl.ds(base, B)])

        _edge_loop(wid, nb, cov, rem_b, do_batch)

    return ek


def _make_agg_kernel(N, E):
    nb, cov, rem_b, rpt, rlast = _partition(N, E)
    f32 = jnp.float32
    mesh = plsc.VectorSubcoreMesh(core_axis_name="c", subcore_axis_name="s")

    @functools.partial(
        pl.kernel,
        out_type=jax.ShapeDtypeStruct((NC, 5, N, 128), f32),
        mesh=mesh,
        scratch_types=[
            pltpu.VMEM((B,), jnp.int32),
            pltpu.VMEM((B,), jnp.int32),
            pltpu.VMEM((B, 128), f32),
            pltpu.VMEM((B, 128), f32),
            pltpu.VMEM_SHARED((N, 128), f32),
            pltpu.SemaphoreType.DMA,
        ],
    )
    def ak(f0, f1, f2, f3, src_h, dst_h, w_h, z_h, raw_h,
           isb, idb, wb, rows, acc, sem):
        cid = lax.axis_index("c")
        sid = lax.axis_index("s")
        wid = sid * NC + cid

        def run_chunk(c, do_batch):
            def zero(off, size):
                pltpu.sync_copy(z_h.at[pl.ds(0, size)], acc.at[pl.ds(off, size)])

            _striped(zero, sid, rpt, rlast)
            plsc.subcore_barrier()
            _edge_loop(wid, nb, cov, rem_b, do_batch)
            plsc.subcore_barrier()

            def copy_out(off, size):
                pltpu.sync_copy(acc.at[pl.ds(off, size)],
                                raw_h.at[cid, c, pl.ds(off, size)])

            _striped(copy_out, sid, rpt, rlast)

        for c, fch in enumerate((f0, f1, f2, f3)):
            def do_batch(base, fch=fch):
                pltpu.sync_copy(src_h.at[pl.ds(base, B)], isb)
                pltpu.sync_copy(dst_h.at[pl.ds(base, B)], idb)
                pltpu.sync_copy(w_h.at[pl.ds(base, B)], wb)
                pltpu.async_copy(fch.at[isb], rows, sem).wait()

                def body(kk, carry):
                    for jj in range(8):
                        sl = pl.ds(jj * 16, 16)
                        rows[kk, sl] = rows[kk, sl] * wb[kk, sl]
                    return carry

                lax.fori_loop(0, B, body, 0)
                pltpu.sync_copy(rows, acc.at[idb], add=True)

            run_chunk(c, do_batch)

        def do_batch_d(base):
            pltpu.sync_copy(dst_h.at[pl.ds(base, B)], idb)
            pltpu.sync_copy(w_h.at[pl.ds(base, B)], wb)
            pltpu.sync_copy(wb, acc.at[idb], add=True)

        run_chunk(4, do_batch_d)

    return ak


def _make_comb_body(permute):
    def body(*args):
        if permute:
            raw_ref, smat_ref, rexp_ref, b_ref, p_ref, out_ref = args
        else:
            raw_ref, smat_ref, rexp_ref, b_ref, out_ref = args
        d = jnp.dot(raw_ref[0, 4] + raw_ref[1, 4], smat_ref[...],
                    preferred_element_type=jnp.float32)
        inv = 1.0 / (d + 1e-9)
        sf = jnp.dot(inv, rexp_ref[...], preferred_element_type=jnp.float32)
        ys = []
        for c in range(4):
            t = (raw_ref[0, c] + raw_ref[1, c]) * sf \
                + b_ref[0, c * 128:(c + 1) * 128]
            y = jnp.where(t > 0, t, jnp.exp(jnp.minimum(t, 0.0)) - 1.0)
            if permute:
                ys.append(y)
            else:
                out_ref[:, c * 128:(c + 1) * 128] = y
        if permute:
            yfull = jnp.concatenate(ys, axis=1)
            out_ref[...] = jnp.dot(yfull, p_ref[...],
                                   preferred_element_type=jnp.float32)
    return body


def _comb_call(raw, smat, rexp, b2d, pmat):
    n = raw.shape[2]
    bm = 1000
    permute = pmat is not None
    in_specs = [
        pl.BlockSpec((NC, 5, bm, 128), lambda i: (0, 0, i, 0)),
        pl.BlockSpec((128, 16), lambda i: (0, 0)),
        pl.BlockSpec((16, 128), lambda i: (0, 0)),
        pl.BlockSpec((1, D), lambda i: (0, 0)),
    ]
    args = [raw, smat, rexp, b2d]
    if permute:
        in_specs.append(pl.BlockSpec((D, D), lambda i: (0, 0)))
        args.append(pmat)
    return pl.pallas_call(
        _make_comb_body(permute),
        grid=(n // bm,),
        in_specs=in_specs,
        out_specs=pl.BlockSpec((bm, D), lambda i: (i, 0)),
        out_shape=jax.ShapeDtypeStruct((n, D), jnp.float32),
    )(*args)


def _perm():
    p = np.zeros((D,), np.int64)
    for c in range(4):
        for h in range(NH):
            for l in range(16):
                p[c * 128 + h * 16 + l] = h * FH + c * 16 + l
    return p


def _attn_mask(perm):
    m = np.zeros((D, 128), np.float32)
    for j in range(D):
        h = perm[j] // FH
        m[j, 16 * h:16 * h + 16] = 1.0
    return m


def _layer(h, src, dst, Wp, aexp_l, aexp_r, b_int, z128, ek, ak, smat, rexp,
           pmat):
    f0, f1, f2, f3, alw, brw = _mm_call(h, Wp, aexp_l, aexp_r)
    w_h = ek(alw, brw, src, dst)
    raw = ak(f0, f1, f2, f3, src, dst, w_h, z128)
    return _comb_call(raw, smat, rexp, b_int, pmat)


def kernel(in_feat, edge_index, W1, attn_l1, attn_r1, b1,
           W2, attn_l2, attn_r2, b2):
    N = in_feat.shape[0]
    E = edge_index.shape[1]
    f32 = jnp.float32
    src = edge_index[0]
    dst = edge_index[1]
    perm = _perm()
    iperm = np.argsort(perm)
    permj = jnp.asarray(perm)
    amask = jnp.asarray(_attn_mask(perm))
    _, _, _, rpt, rlast = _partition(N, E)
    z128 = jnp.zeros((rlast, 128), f32)
    smat = jnp.zeros((128, 16), f32).at[
        16 * jnp.arange(NH), jnp.arange(NH)].set(1.0)
    rx = np.zeros((16, 128), np.float32)
    for h in range(NH):
        rx[h, 16 * h:16 * h + 16] = 1.0
    rexp = jnp.asarray(rx)
    pm = np.zeros((D, D), np.float32)
    pm[iperm, np.arange(D)] = 1.0
    pmat = jnp.asarray(pm)
    ek = _make_edge_kernel(N, E)
    ak = _make_agg_kernel(N, E)

    def prep(W, attn_l, attn_r, b, permute_rows):
        Wp = W[permj, :] if permute_rows else W
        Wp = Wp[:, permj]
        aexp_l = amask * attn_l.reshape(-1)[permj][:, None]
        aexp_r = amask * attn_r.reshape(-1)[permj][:, None]
        return Wp, aexp_l, aexp_r, b[permj].reshape(1, -1)

    W1p, al1, ar1, b1i = prep(W1, attn_l1, attn_r1, b1, False)
    W2p, al2, ar2, b2i = prep(W2, attn_l2, attn_r2, b2, True)
    h1 = _layer(in_feat, src, dst, W1p, al1, ar1, b1i, z128, ek, ak,
                smat, rexp, None)
    h2 = _layer(h1, src, dst, W2p, al2, ar2, b2i, z128, ek, ak,
                smat, rexp, pmat)
    return h2


# speedup vs baseline: 19.5439x; 1.2430x over previous
import functools

import jax
import jax.numpy as jnp
import numpy as np
from jax import lax
from jax.experimental import pallas as pl
from jax.experimental.pallas import tpu as pltpu
from jax.experimental.pallas import tpu_sc as plsc

NC = 2
NS = 16
B = 128
NH = 8
FH = 64
D = NH * FH


def _mm_body(x_ref, w_ref, al_ref, ar_ref, f0, f1, f2, f3, alo, bro):
    feat = jnp.dot(x_ref[...], w_ref[...], preferred_element_type=jnp.float32)
    f0[...] = feat[:, 0:128]
    f1[...] = feat[:, 128:256]
    f2[...] = feat[:, 256:384]
    f3[...] = feat[:, 384:512]
    alo[...] = jnp.dot(feat, al_ref[...], preferred_element_type=jnp.float32)
    bro[...] = jnp.dot(feat, ar_ref[...], preferred_element_type=jnp.float32)


def _mm_call(x, Wm, Aexp_l, Aexp_r):
    n, k = x.shape
    bm = 1000
    f32 = jnp.float32
    return pl.pallas_call(
        _mm_body,
        grid=(n // bm,),
        in_specs=[
            pl.BlockSpec((bm, k), lambda i: (i, 0)),
            pl.BlockSpec((k, D), lambda i: (0, 0)),
            pl.BlockSpec((D, 128), lambda i: (0, 0)),
            pl.BlockSpec((D, 128), lambda i: (0, 0)),
        ],
        out_specs=[pl.BlockSpec((bm, 128), lambda i: (i, 0))] * 6,
        out_shape=[jax.ShapeDtypeStruct((n, 128), f32)] * 6,
    )(x, Wm, Aexp_l, Aexp_r)


def _partition(N, E):
    nb = E // (NC * NS * B)
    cov = NC * NS * B * nb
    rem_b = (E - cov) // B
    assert cov + rem_b * B == E
    rpt = (N // NS) // 8 * 8
    rlast = N - rpt * (NS - 1)
    return nb, cov, rem_b, rpt, rlast


def _striped(copy_fn, sid, rpt, rlast):
    @pl.when(sid < NS - 1)
    def _():
        copy_fn(sid * rpt, rpt)

    @pl.when(sid == NS - 1)
    def _():
        copy_fn((NS - 1) * rpt, rlast)


def _make_edge_kernel(N, E):
    nb, cov, rem_b, rpt, rlast = _partition(N, E)
    f32 = jnp.float32
    mesh = plsc.VectorSubcoreMesh(core_axis_name="c", subcore_axis_name="s")

    @functools.partial(
        pl.kernel,
        out_type=jax.ShapeDtypeStruct((E, 128), f32),
        mesh=mesh,
        scratch_types=[
            pltpu.VMEM((B,), jnp.int32),
            pltpu.VMEM((B,), jnp.int32),
            pltpu.VMEM((B,), jnp.int32),
            pltpu.VMEM((B,), jnp.int32),
            pltpu.VMEM((B, 128), f32),
            pltpu.VMEM((B, 128), f32),
            pltpu.VMEM((B, 128), f32),
            pltpu.VMEM((B, 128), f32),
            pltpu.SemaphoreType.DMA,
            pltpu.SemaphoreType.DMA,
            pltpu.SemaphoreType.DMA,
            pltpu.SemaphoreType.DMA,
        ],
    )
    def ek(al_h, br_h, src_h, dst_h, w_h,
           is0, id0, is1, id1, ga0, gb0, ga1, gb1,
           sa0, sb0, sa1, sb1):
        cid = lax.axis_index("c")
        sid = lax.axis_index("s")
        wid = sid * NC + cid
        base_w = wid * (nb * B)

        def prefetch(base, isb, idb, ga, gb, sa, sb):
            pltpu.sync_copy(src_h.at[pl.ds(base, B)], isb)
            pltpu.sync_copy(dst_h.at[pl.ds(base, B)], idb)
            pltpu.async_copy(al_h.at[isb], ga, sa)
            pltpu.async_copy(br_h.at[idb], gb, sb)

        def compute_store(base, isb, idb, ga, gb, sa, sb):
            pltpu.make_async_copy(al_h.at[isb], ga, sa).wait()
            pltpu.make_async_copy(br_h.at[idb], gb, sb).wait()

            def body(kk, carry):
                for jj in range(8):
                    sl = pl.ds(jj * 16, 16)
                    x = ga[kk, sl] + gb[kk, sl]
                    ga[kk, sl] = jnp.exp(jnp.maximum(x, 0.2 * x))
                return carry

            lax.fori_loop(0, B, body, 0)
            pltpu.sync_copy(ga, w_h.at[pl.ds(base, B)])

        assert nb >= 3 and nb % 2 == 1
        npair = (nb - 1) // 2
        prefetch(base_w, is0, id0, ga0, gb0, sa0, sb0)

        def pair(p, carry):
            a = base_w + (2 * p) * B
            bbt = a + B
            nxt = a + 2 * B
            prefetch(bbt, is1, id1, ga1, gb1, sa1, sb1)
            compute_store(a, is0, id0, ga0, gb0, sa0, sb0)
            prefetch(nxt, is0, id0, ga0, gb0, sa0, sb0)
            compute_store(bbt, is1, id1, ga1, gb1, sa1, sb1)
            return carry

        lax.fori_loop(0, npair, pair, 0)
        compute_store(base_w + (nb - 1) * B, is0, id0, ga0, gb0, sa0, sb0)
        if rem_b:
            @pl.when(wid < rem_b)
            def _():
                prefetch(cov + wid * B, is1, id1, ga1, gb1, sa1, sb1)
                compute_store(cov + wid * B, is1, id1, ga1, gb1, sa1, sb1)

    return ek


def _make_agg_kernel(N, E):
    nb, cov, rem_b, rpt, rlast = _partition(N, E)
    f32 = jnp.float32
    mesh = plsc.VectorSubcoreMesh(core_axis_name="c", subcore_axis_name="s")

    @functools.partial(
        pl.kernel,
        out_type=jax.ShapeDtypeStruct((NC, 5, N, 128), f32),
        mesh=mesh,
        scratch_types=[
            pltpu.VMEM((B,), jnp.int32),
            pltpu.VMEM((B,), jnp.int32),
            pltpu.VMEM((B,), jnp.int32),
            pltpu.VMEM((B, 128), f32),
            pltpu.VMEM((B, 128), f32),
            pltpu.VMEM((B, 128), f32),
            pltpu.VMEM_SHARED((N, 128), f32),
            pltpu.SemaphoreType.DMA,
            pltpu.SemaphoreType.DMA,
        ],
    )
    def ak(f0, f1, f2, f3, src_h, dst_h, w_h, z_h, raw_h,
           is0, is1, id0, wb0, rows0, rows1, acc,
           sg0, sg1):
        cid = lax.axis_index("c")
        sid = lax.axis_index("s")
        wid = sid * NC + cid
        base_w = wid * (nb * B)
        assert nb >= 3 and nb % 2 == 1
        npair = (nb - 1) // 2

        def run_chunk(c, fch):
            def zero(off, size):
                pltpu.sync_copy(z_h.at[pl.ds(0, size)], acc.at[pl.ds(off, size)])

            _striped(zero, sid, rpt, rlast)
            plsc.subcore_barrier()

            def prefetch(base, isb, rows, sg):
                if fch is not None:
                    pltpu.sync_copy(src_h.at[pl.ds(base, B)], isb)
                    pltpu.async_copy(fch.at[isb], rows, sg)

            def compute_store(base, isb, rows, sg):
                pltpu.sync_copy(dst_h.at[pl.ds(base, B)], id0)
                pltpu.sync_copy(w_h.at[pl.ds(base, B)], wb0)
                if fch is None:
                    pltpu.sync_copy(wb0, acc.at[id0], add=True)
                    return
                pltpu.make_async_copy(fch.at[isb], rows, sg).wait()

                def body(kk, carry):
                    for jj in range(8):
                        sl = pl.ds(jj * 16, 16)
                        rows[kk, sl] = rows[kk, sl] * wb0[kk, sl]
                    return carry

                lax.fori_loop(0, B, body, 0)
                pltpu.sync_copy(rows, acc.at[id0], add=True)

            prefetch(base_w, is0, rows0, sg0)

            def pair(p, carry):
                a = base_w + (2 * p) * B
                prefetch(a + B, is1, rows1, sg1)
                compute_store(a, is0, rows0, sg0)
                prefetch(a + 2 * B, is0, rows0, sg0)
                compute_store(a + B, is1, rows1, sg1)
                return carry

            lax.fori_loop(0, npair, pair, 0)
            compute_store(base_w + (nb - 1) * B, is0, rows0, sg0)
            if rem_b:
                @pl.when(wid < rem_b)
                def _():
                    prefetch(cov + wid * B, is1, rows1, sg1)
                    compute_store(cov + wid * B, is1, rows1, sg1)
            plsc.subcore_barrier()

            def copy_out(off, size):
                pltpu.sync_copy(acc.at[pl.ds(off, size)],
                                raw_h.at[cid, c, pl.ds(off, size)])

            _striped(copy_out, sid, rpt, rlast)

        for c, fch in enumerate((f0, f1, f2, f3)):
            run_chunk(c, fch)
        run_chunk(4, None)

    return ak


def _make_comb_body(permute):
    def body(*args):
        if permute:
            raw_ref, smat_ref, rexp_ref, b_ref, p_ref, out_ref = args
        else:
            raw_ref, smat_ref, rexp_ref, b_ref, out_ref = args
        d = jnp.dot(raw_ref[0, 4] + raw_ref[1, 4], smat_ref[...],
                    preferred_element_type=jnp.float32)
        inv = 1.0 / (d + 1e-9)
        sf = jnp.dot(inv, rexp_ref[...], preferred_element_type=jnp.float32)
        ys = []
        for c in range(4):
            t = (raw_ref[0, c] + raw_ref[1, c]) * sf \
                + b_ref[0, c * 128:(c + 1) * 128]
            y = jnp.where(t > 0, t, jnp.exp(jnp.minimum(t, 0.0)) - 1.0)
            if permute:
                ys.append(y)
            else:
                out_ref[:, c * 128:(c + 1) * 128] = y
        if permute:
            yfull = jnp.concatenate(ys, axis=1)
            out_ref[...] = jnp.dot(yfull, p_ref[...],
                                   preferred_element_type=jnp.float32)
    return body


def _comb_call(raw, smat, rexp, b2d, pmat):
    n = raw.shape[2]
    bm = 1000
    permute = pmat is not None
    in_specs = [
        pl.BlockSpec((NC, 5, bm, 128), lambda i: (0, 0, i, 0)),
        pl.BlockSpec((128, 16), lambda i: (0, 0)),
        pl.BlockSpec((16, 128), lambda i: (0, 0)),
        pl.BlockSpec((1, D), lambda i: (0, 0)),
    ]
    args = [raw, smat, rexp, b2d]
    if permute:
        in_specs.append(pl.BlockSpec((D, D), lambda i: (0, 0)))
        args.append(pmat)
    return pl.pallas_call(
        _make_comb_body(permute),
        grid=(n // bm,),
        in_specs=in_specs,
        out_specs=pl.BlockSpec((bm, D), lambda i: (i, 0)),
        out_shape=jax.ShapeDtypeStruct((n, D), jnp.float32),
    )(*args)


def _perm():
    p = np.zeros((D,), np.int64)
    for c in range(4):
        for h in range(NH):
            for l in range(16):
                p[c * 128 + h * 16 + l] = h * FH + c * 16 + l
    return p


def _attn_mask(perm):
    m = np.zeros((D, 128), np.float32)
    for j in range(D):
        h = perm[j] // FH
        m[j, 16 * h:16 * h + 16] = 1.0
    return m


def _layer(h, src, dst, Wp, aexp_l, aexp_r, b_int, z128, ek, ak, smat, rexp,
           pmat):
    f0, f1, f2, f3, alw, brw = _mm_call(h, Wp, aexp_l, aexp_r)
    w_h = ek(alw, brw, src, dst)
    raw = ak(f0, f1, f2, f3, src, dst, w_h, z128)
    return _comb_call(raw, smat, rexp, b_int, pmat)


def kernel(in_feat, edge_index, W1, attn_l1, attn_r1, b1,
           W2, attn_l2, attn_r2, b2):
    N = in_feat.shape[0]
    E = edge_index.shape[1]
    f32 = jnp.float32
    src = edge_index[0]
    dst = edge_index[1]
    perm = _perm()
    iperm = np.argsort(perm)
    permj = jnp.asarray(perm)
    amask = jnp.asarray(_attn_mask(perm))
    _, _, _, rpt, rlast = _partition(N, E)
    z128 = jnp.zeros((rlast, 128), f32)
    smat = jnp.zeros((128, 16), f32).at[
        16 * jnp.arange(NH), jnp.arange(NH)].set(1.0)
    rx = np.zeros((16, 128), np.float32)
    for h in range(NH):
        rx[h, 16 * h:16 * h + 16] = 1.0
    rexp = jnp.asarray(rx)
    pm = np.zeros((D, D), np.float32)
    pm[iperm, np.arange(D)] = 1.0
    pmat = jnp.asarray(pm)
    ek = _make_edge_kernel(N, E)
    ak = _make_agg_kernel(N, E)

    def prep(W, attn_l, attn_r, b, permute_rows):
        Wp = W[permj, :] if permute_rows else W
        Wp = Wp[:, permj]
        aexp_l = amask * attn_l.reshape(-1)[permj][:, None]
        aexp_r = amask * attn_r.reshape(-1)[permj][:, None]
        return Wp, aexp_l, aexp_r, b[permj].reshape(1, -1)

    W1p, al1, ar1, b1i = prep(W1, attn_l1, attn_r1, b1, False)
    W2p, al2, ar2, b2i = prep(W2, attn_l2, attn_r2, b2, True)
    h1 = _layer(in_feat, src, dst, W1p, al1, ar1, b1i, z128, ek, ak,
                smat, rexp, None)
    h2 = _layer(h1, src, dst, W2p, al2, ar2, b2i, z128, ek, ak,
                smat, rexp, pmat)
    return h2


# speedup vs baseline: 20.4267x; 1.0452x over previous
import functools

import jax
import jax.numpy as jnp
import numpy as np
from jax import lax
from jax.experimental import pallas as pl
from jax.experimental.pallas import tpu as pltpu
from jax.experimental.pallas import tpu_sc as plsc

NC = 2
NS = 16
B = 128
NH = 8
FH = 64
D = NH * FH


def _mm_body(x_ref, w_ref, al_ref, ar_ref, f0, f1, f2, f3, alo, bro):
    feat = jnp.dot(x_ref[...], w_ref[...], preferred_element_type=jnp.float32)
    f0[...] = feat[:, 0:128]
    f1[...] = feat[:, 128:256]
    f2[...] = feat[:, 256:384]
    f3[...] = feat[:, 384:512]
    alo[...] = jnp.dot(feat, al_ref[...], preferred_element_type=jnp.float32)
    bro[...] = jnp.dot(feat, ar_ref[...], preferred_element_type=jnp.float32)


def _mm_call(x, Wm, Aexp_l, Aexp_r):
    n, k = x.shape
    bm = 1000
    f32 = jnp.float32
    return pl.pallas_call(
        _mm_body,
        grid=(n // bm,),
        in_specs=[
            pl.BlockSpec((bm, k), lambda i: (i, 0)),
            pl.BlockSpec((k, D), lambda i: (0, 0)),
            pl.BlockSpec((D, 128), lambda i: (0, 0)),
            pl.BlockSpec((D, 128), lambda i: (0, 0)),
        ],
        out_specs=[pl.BlockSpec((bm, 128), lambda i: (i, 0))] * 6,
        out_shape=[jax.ShapeDtypeStruct((n, 128), f32)] * 6,
    )(x, Wm, Aexp_l, Aexp_r)


def _partition(N, E):
    nb = E // (NC * NS * B)
    cov = NC * NS * B * nb
    rem_b = (E - cov) // B
    assert cov + rem_b * B == E
    rpt = (N // NS) // 8 * 8
    rlast = N - rpt * (NS - 1)
    return nb, cov, rem_b, rpt, rlast


def _striped(copy_fn, sid, rpt, rlast):
    @pl.when(sid < NS - 1)
    def _():
        copy_fn(sid * rpt, rpt)

    @pl.when(sid == NS - 1)
    def _():
        copy_fn((NS - 1) * rpt, rlast)


def _make_edge_kernel(N, E):
    nb, cov, rem_b, rpt, rlast = _partition(N, E)
    f32 = jnp.float32
    mesh = plsc.VectorSubcoreMesh(core_axis_name="c", subcore_axis_name="s")

    @functools.partial(
        pl.kernel,
        out_type=jax.ShapeDtypeStruct((E, 128), f32),
        mesh=mesh,
        scratch_types=[
            pltpu.VMEM((B,), jnp.int32),
            pltpu.VMEM((B,), jnp.int32),
            pltpu.VMEM((B,), jnp.int32),
            pltpu.VMEM((B,), jnp.int32),
            pltpu.VMEM((B, 128), f32),
            pltpu.VMEM((B, 128), f32),
            pltpu.VMEM((B, 128), f32),
            pltpu.VMEM((B, 128), f32),
            pltpu.SemaphoreType.DMA,
            pltpu.SemaphoreType.DMA,
            pltpu.SemaphoreType.DMA,
            pltpu.SemaphoreType.DMA,
        ],
    )
    def ek(al_h, br_h, src_h, dst_h, w_h,
           is0, id0, is1, id1, ga0, gb0, ga1, gb1,
           sa0, sb0, sa1, sb1):
        cid = lax.axis_index("c")
        sid = lax.axis_index("s")
        wid = sid * NC + cid
        base_w = wid * (nb * B)

        def prefetch(base, isb, idb, ga, gb, sa, sb):
            pltpu.sync_copy(src_h.at[pl.ds(base, B)], isb)
            pltpu.sync_copy(dst_h.at[pl.ds(base, B)], idb)
            pltpu.async_copy(al_h.at[isb], ga, sa)
            pltpu.async_copy(br_h.at[idb], gb, sb)

        def compute_store(base, isb, idb, ga, gb, sa, sb):
            pltpu.make_async_copy(al_h.at[isb], ga, sa).wait()
            pltpu.make_async_copy(br_h.at[idb], gb, sb).wait()

            def body(kk, carry):
                for jj in range(8):
                    sl = pl.ds(jj * 16, 16)
                    x = ga[kk, sl] + gb[kk, sl]
                    ga[kk, sl] = jnp.exp(jnp.maximum(x, 0.2 * x))
                return carry

            lax.fori_loop(0, B, body, 0)
            pltpu.sync_copy(ga, w_h.at[pl.ds(base, B)])

        assert nb >= 3 and nb % 2 == 1
        npair = (nb - 1) // 2
        prefetch(base_w, is0, id0, ga0, gb0, sa0, sb0)

        def pair(p, carry):
            a = base_w + (2 * p) * B
            bbt = a + B
            nxt = a + 2 * B
            prefetch(bbt, is1, id1, ga1, gb1, sa1, sb1)
            compute_store(a, is0, id0, ga0, gb0, sa0, sb0)
            prefetch(nxt, is0, id0, ga0, gb0, sa0, sb0)
            compute_store(bbt, is1, id1, ga1, gb1, sa1, sb1)
            return carry

        lax.fori_loop(0, npair, pair, 0)
        compute_store(base_w + (nb - 1) * B, is0, id0, ga0, gb0, sa0, sb0)
        if rem_b:
            @pl.when(wid < rem_b)
            def _():
                prefetch(cov + wid * B, is1, id1, ga1, gb1, sa1, sb1)
                compute_store(cov + wid * B, is1, id1, ga1, gb1, sa1, sb1)

    return ek


def _make_agg_kernel(N, E):
    nb, cov, rem_b, rpt, rlast = _partition(N, E)
    f32 = jnp.float32
    mesh = plsc.VectorSubcoreMesh(core_axis_name="c", subcore_axis_name="s")

    @functools.partial(
        pl.kernel,
        out_type=jax.ShapeDtypeStruct((NC, 5, N, 128), f32),
        mesh=mesh,
        scratch_types=[
            pltpu.VMEM((B,), jnp.int32),
            pltpu.VMEM((B,), jnp.int32),
            pltpu.VMEM((B,), jnp.int32),
            pltpu.VMEM((B, 128), f32),
            pltpu.VMEM((B, 128), f32),
            pltpu.VMEM((B, 128), f32),
            pltpu.VMEM_SHARED((N, 128), f32),
            pltpu.SemaphoreType.DMA,
            pltpu.SemaphoreType.DMA,
        ],
    )
    def ak(f0, f1, f2, f3, src_h, dst_h, w_h, z_h, raw_h,
           is0, is1, id0, wb0, rows0, rows1, acc,
           sg0, sg1):
        cid = lax.axis_index("c")
        sid = lax.axis_index("s")
        wid = sid * NC + cid
        base_w = wid * (nb * B)
        assert nb >= 3 and nb % 2 == 1
        npair = (nb - 1) // 2

        def run_chunk(c, fch):
            def zero(off, size):
                pltpu.sync_copy(z_h.at[pl.ds(0, size)], acc.at[pl.ds(off, size)])

            _striped(zero, sid, rpt, rlast)
            plsc.subcore_barrier()

            def prefetch(base, isb, rows, sg):
                if fch is not None:
                    pltpu.sync_copy(src_h.at[pl.ds(base, B)], isb)
                    pltpu.async_copy(fch.at[isb], rows, sg)
                else:
                    pltpu.async_copy(w_h.at[pl.ds(base, B)], rows, sg)

            def compute_store(base, isb, rows, sg):
                pltpu.sync_copy(dst_h.at[pl.ds(base, B)], id0)
                if fch is None:
                    pltpu.make_async_copy(w_h.at[pl.ds(base, B)], rows,
                                          sg).wait()
                    pltpu.sync_copy(rows, acc.at[id0], add=True)
                    return
                pltpu.sync_copy(w_h.at[pl.ds(base, B)], wb0)
                pltpu.make_async_copy(fch.at[isb], rows, sg).wait()

                def body(kk, carry):
                    for jj in range(8):
                        sl = pl.ds(jj * 16, 16)
                        rows[kk, sl] = rows[kk, sl] * wb0[kk, sl]
                    return carry

                lax.fori_loop(0, B, body, 0)
                pltpu.sync_copy(rows, acc.at[id0], add=True)

            prefetch(base_w, is0, rows0, sg0)

            def pair(p, carry):
                a = base_w + (2 * p) * B
                prefetch(a + B, is1, rows1, sg1)
                compute_store(a, is0, rows0, sg0)
                prefetch(a + 2 * B, is0, rows0, sg0)
                compute_store(a + B, is1, rows1, sg1)
                return carry

            lax.fori_loop(0, npair, pair, 0)
            compute_store(base_w + (nb - 1) * B, is0, rows0, sg0)
            if rem_b:
                @pl.when(wid < rem_b)
                def _():
                    prefetch(cov + wid * B, is1, rows1, sg1)
                    compute_store(cov + wid * B, is1, rows1, sg1)
            plsc.subcore_barrier()

            def copy_out(off, size):
                pltpu.sync_copy(acc.at[pl.ds(off, size)],
                                raw_h.at[cid, c, pl.ds(off, size)])

            _striped(copy_out, sid, rpt, rlast)

        for c, fch in enumerate((f0, f1, f2, f3)):
            run_chunk(c, fch)
        run_chunk(4, None)

    return ak


def _make_comb_body(permute):
    def body(*args):
        if permute:
            raw_ref, smat_ref, rexp_ref, b_ref, p_ref, out_ref = args
        else:
            raw_ref, smat_ref, rexp_ref, b_ref, out_ref = args
        d = jnp.dot(raw_ref[0, 4] + raw_ref[1, 4], smat_ref[...],
                    preferred_element_type=jnp.float32)
        inv = 1.0 / (d + 1e-9)
        sf = jnp.dot(inv, rexp_ref[...], preferred_element_type=jnp.float32)
        ys = []
        for c in range(4):
            t = (raw_ref[0, c] + raw_ref[1, c]) * sf \
                + b_ref[0, c * 128:(c + 1) * 128]
            y = jnp.where(t > 0, t, jnp.exp(jnp.minimum(t, 0.0)) - 1.0)
            if permute:
                ys.append(y)
            else:
                out_ref[:, c * 128:(c + 1) * 128] = y
        if permute:
            yfull = jnp.concatenate(ys, axis=1)
            out_ref[...] = jnp.dot(yfull, p_ref[...],
                                   preferred_element_type=jnp.float32)
    return body


def _comb_call(raw, smat, rexp, b2d, pmat):
    n = raw.shape[2]
    bm = 1000
    permute = pmat is not None
    in_specs = [
        pl.BlockSpec((NC, 5, bm, 128), lambda i: (0, 0, i, 0)),
        pl.BlockSpec((128, 16), lambda i: (0, 0)),
        pl.BlockSpec((16, 128), lambda i: (0, 0)),
        pl.BlockSpec((1, D), lambda i: (0, 0)),
    ]
    args = [raw, smat, rexp, b2d]
    if permute:
        in_specs.append(pl.BlockSpec((D, D), lambda i: (0, 0)))
        args.append(pmat)
    return pl.pallas_call(
        _make_comb_body(permute),
        grid=(n // bm,),
        in_specs=in_specs,
        out_specs=pl.BlockSpec((bm, D), lambda i: (i, 0)),
        out_shape=jax.ShapeDtypeStruct((n, D), jnp.float32),
    )(*args)


def _perm():
    p = np.zeros((D,), np.int64)
    for c in range(4):
        for h in range(NH):
            for l in range(16):
                p[c * 128 + h * 16 + l] = h * FH + c * 16 + l
    return p


def _attn_mask(perm):
    m = np.zeros((D, 128), np.float32)
    for j in range(D):
        h = perm[j] // FH
        m[j, 16 * h:16 * h + 16] = 1.0
    return m


def _layer(h, src, dst, Wp, aexp_l, aexp_r, b_int, z128, ek, ak, smat, rexp,
           pmat):
    f0, f1, f2, f3, alw, brw = _mm_call(h, Wp, aexp_l, aexp_r)
    w_h = ek(alw, brw, src, dst)
    raw = ak(f0, f1, f2, f3, src, dst, w_h, z128)
    return _comb_call(raw, smat, rexp, b_int, pmat)


def kernel(in_feat, edge_index, W1, attn_l1, attn_r1, b1,
           W2, attn_l2, attn_r2, b2):
    N = in_feat.shape[0]
    E = edge_index.shape[1]
    f32 = jnp.float32
    src = edge_index[0]
    dst = edge_index[1]
    perm = _perm()
    iperm = np.argsort(perm)
    permj = jnp.asarray(perm)
    amask = jnp.asarray(_attn_mask(perm))
    _, _, _, rpt, rlast = _partition(N, E)
    z128 = jnp.zeros((rlast, 128), f32)
    smat = jnp.zeros((128, 16), f32).at[
        16 * jnp.arange(NH), jnp.arange(NH)].set(1.0)
    rx = np.zeros((16, 128), np.float32)
    for h in range(NH):
        rx[h, 16 * h:16 * h + 16] = 1.0
    rexp = jnp.asarray(rx)
    pm = np.zeros((D, D), np.float32)
    pm[iperm, np.arange(D)] = 1.0
    pmat = jnp.asarray(pm)
    ek = _make_edge_kernel(N, E)
    ak = _make_agg_kernel(N, E)

    def prep(W, attn_l, attn_r, b, permute_rows):
        Wp = W[permj, :] if permute_rows else W
        Wp = Wp[:, permj]
        aexp_l = amask * attn_l.reshape(-1)[permj][:, None]
        aexp_r = amask * attn_r.reshape(-1)[permj][:, None]
        return Wp, aexp_l, aexp_r, b[permj].reshape(1, -1)

    W1p, al1, ar1, b1i = prep(W1, attn_l1, attn_r1, b1, False)
    W2p, al2, ar2, b2i = prep(W2, attn_l2, attn_r2, b2, True)
    h1 = _layer(in_feat, src, dst, W1p, al1, ar1, b1i, z128, ek, ak,
                smat, rexp, None)
    h2 = _layer(h1, src, dst, W2p, al2, ar2, b2i, z128, ek, ak,
                smat, rexp, pmat)
    return h2
